# baseline (device time: 364643 ns/iter reference)
import jax
import jax.numpy as jnp
from jax import lax
from jax.experimental import pallas as pl
from jax.experimental.pallas import tpu as pltpu

N_DEV = 8
N_STEP = 2 * (N_DEV - 1)
K_SUB = 2


def kernel(x, w_mat):
    m, k_per = x.shape
    _, n = w_mat.shape
    halfm = m // 2
    assert halfm % N_DEV == 0
    hchunk = halfm // N_DEV
    assert hchunk % K_SUB == 0
    sub = hchunk // K_SUB

    def body(
        x_ref, w_ref, out_ref,
        recv_a, recv_b,
        send_sem_a, recv_sem_a, send_sem_b, recv_sem_b,
        ready_a, ready_b,
    ):
        my = lax.axis_index("i")
        left = lax.rem(my - 1 + N_DEV, N_DEV)
        right = lax.rem(my + 1, N_DEV)

        def rows_a(c, k):
            return pl.ds(c * hchunk + k * sub, sub)

        def rows_b(c, k):
            return pl.ds(halfm + c * hchunk + k * sub, sub)

        def gemm_half_a(c):
            ia = pl.ds(c * hchunk, hchunk)
            out_ref[ia, :] = jnp.dot(
                x_ref[ia, :], w_ref[:, :], preferred_element_type=jnp.float32
            )

        def gemm_half_b(c):
            ib = pl.ds(halfm + c * hchunk, hchunk)
            out_ref[ib, :] = jnp.dot(
                x_ref[ib, :], w_ref[:, :], preferred_element_type=jnp.float32
            )

        def chunk_ids(s):
            if s < N_DEV - 1:
                return (
                    lax.rem(my - s + N_DEV, N_DEV),
                    lax.rem(my - s - 1 + 2 * N_DEV, N_DEV),
                    lax.rem(my + s, N_DEV),
                    lax.rem(my + s + 1, N_DEV),
                )
            t = s - (N_DEV - 1)
            return (
                lax.rem(my + 1 - t + 2 * N_DEV, N_DEV),
                lax.rem(my - t + 2 * N_DEV, N_DEV),
                lax.rem(my - 1 + t + N_DEV, N_DEV),
                lax.rem(my + t, N_DEV),
            )

        def issue(s, k):
            slot = s % 2
            ag = s >= N_DEV - 1
            send_a_c, _, send_b_c, _ = chunk_ids(s)
            dst_a = recv_a.at[slot, k] if not ag else out_ref.at[rows_a(send_a_c, k), :]
            dst_b = recv_b.at[slot, k] if not ag else out_ref.at[rows_b(send_b_c, k), :]
            rdma_a = pltpu.make_async_remote_copy(
                src_ref=out_ref.at[rows_a(send_a_c, k), :],
                dst_ref=dst_a,
                send_sem=send_sem_a.at[slot, k],
                recv_sem=recv_sem_a.at[slot, k],
                device_id=(right,),
                device_id_type=pl.DeviceIdType.MESH,
            )
            rdma_b = pltpu.make_async_remote_copy(
                src_ref=out_ref.at[rows_b(send_b_c, k), :],
                dst_ref=dst_b,
                send_sem=send_sem_b.at[slot, k],
                recv_sem=recv_sem_b.at[slot, k],
                device_id=(left,),
                device_id_type=pl.DeviceIdType.MESH,
            )
            rdma_a.start()
            rdma_b.start()
            return rdma_a, rdma_b

        def grant(inc=1):
            pl.semaphore_signal(
                ready_a, inc=inc,
                device_id=(left,), device_id_type=pl.DeviceIdType.MESH,
            )
            pl.semaphore_signal(
                ready_b, inc=inc,
                device_id=(right,), device_id_type=pl.DeviceIdType.MESH,
            )

        grant(inc=2)

        gemm_half_a(my)
        gemm_half_b(my)
        pl.semaphore_wait(ready_a, 1)
        pl.semaphore_wait(ready_b, 1)
        inflight = {(0, k): issue(0, k) for k in range(K_SUB)}

        for s in range(N_STEP):
            ag = s >= N_DEV - 1
            _, recv_a_c, _, recv_b_c = chunk_ids(s)
            if not ag:
                gemm_half_a(recv_a_c)
                gemm_half_b(recv_b_c)
            if s + 1 < N_STEP:
                pl.semaphore_wait(ready_a, 1)
                pl.semaphore_wait(ready_b, 1)
            for k in range(K_SUB):
                rdma_a, rdma_b = inflight[(s, k)]
                rdma_a.wait_recv()
                if not ag:
                    ia = rows_a(recv_a_c, k)
                    out_ref[ia, :] = out_ref[ia, :] + recv_a[s % 2, k, :, :]
                rdma_b.wait_recv()
                if not ag:
                    ib = rows_b(recv_b_c, k)
                    out_ref[ib, :] = out_ref[ib, :] + recv_b[s % 2, k, :, :]
                if s + 1 < N_STEP:
                    if s >= 1:
                        pa, pb = inflight.pop((s - 1, k))
                        pa.wait_send()
                        pb.wait_send()
                    inflight[(s + 1, k)] = issue(s + 1, k)
            if s < N_STEP - 2:
                grant()

        for key in sorted(inflight):
            pa, pb = inflight.pop(key)
            pa.wait_send()
            pb.wait_send()

    return pl.pallas_call(
        body,
        out_shape=jax.ShapeDtypeStruct((m, n), jnp.float32),
        in_specs=[
            pl.BlockSpec(memory_space=pltpu.VMEM),
            pl.BlockSpec(memory_space=pltpu.VMEM),
        ],
        out_specs=pl.BlockSpec(memory_space=pltpu.VMEM),
        scratch_shapes=[
            pltpu.VMEM((2, K_SUB, sub, n), jnp.float32),
            pltpu.VMEM((2, K_SUB, sub, n), jnp.float32),
            pltpu.SemaphoreType.DMA((2, K_SUB)),
            pltpu.SemaphoreType.DMA((2, K_SUB)),
            pltpu.SemaphoreType.DMA((2, K_SUB)),
            pltpu.SemaphoreType.DMA((2, K_SUB)),
            pltpu.SemaphoreType.REGULAR,
            pltpu.SemaphoreType.REGULAR,
        ],
        compiler_params=pltpu.CompilerParams(
            vmem_limit_bytes=100 * 1024 * 1024,
        ),
    )(x, w_mat)


# device time: 353679 ns/iter; 1.0310x vs baseline; 1.0310x over previous
import jax
import jax.numpy as jnp
from jax import lax
from jax.experimental import pallas as pl
from jax.experimental.pallas import tpu as pltpu

N_DEV = 8
N_STEP = 2 * (N_DEV - 1)
K_SUB = 2


def kernel(x, w_mat):
    m, k_per = x.shape
    _, n = w_mat.shape
    halfm = m // 2
    assert halfm % N_DEV == 0
    hchunk = halfm // N_DEV
    assert hchunk % K_SUB == 0
    sub = hchunk // K_SUB

    def body(
        x_ref, w_ref, out_ref,
        acc, recv_a, recv_b,
        send_sem_a, recv_sem_a, send_sem_b, recv_sem_b,
        ready_a, ready_b, own_sem,
    ):
        my = lax.axis_index("i")
        left = lax.rem(my - 1 + N_DEV, N_DEV)
        right = lax.rem(my + 1, N_DEV)

        def rows_a(c, k):
            return pl.ds(c * hchunk + k * sub, sub)

        def rows_b(c, k):
            return pl.ds(halfm + c * hchunk + k * sub, sub)

        def gemm_half_a(c):
            ia = pl.ds(c * hchunk, hchunk)
            acc[ia, :] = jnp.dot(
                x_ref[ia, :], w_ref[:, :], preferred_element_type=jnp.float32
            )

        def gemm_half_b(c):
            ib = pl.ds(halfm + c * hchunk, hchunk)
            acc[ib, :] = jnp.dot(
                x_ref[ib, :], w_ref[:, :], preferred_element_type=jnp.float32
            )

        def chunk_ids(s):
            if s < N_DEV - 1:
                return (
                    lax.rem(my - s + N_DEV, N_DEV),
                    lax.rem(my - s - 1 + 2 * N_DEV, N_DEV),
                    lax.rem(my + s, N_DEV),
                    lax.rem(my + s + 1, N_DEV),
                )
            t = s - (N_DEV - 1)
            return (
                lax.rem(my + 1 - t + 2 * N_DEV, N_DEV),
                lax.rem(my - t + 2 * N_DEV, N_DEV),
                lax.rem(my - 1 + t + N_DEV, N_DEV),
                lax.rem(my + t, N_DEV),
            )

        def issue(s, k):
            slot = s % 2
            send_a_c, _, send_b_c, _ = chunk_ids(s)
            if s < N_DEV - 1:
                src_a = acc.at[rows_a(send_a_c, k), :]
                src_b = acc.at[rows_b(send_b_c, k), :]
                dst_a = recv_a.at[slot, k]
                dst_b = recv_b.at[slot, k]
            else:
                if s == N_DEV - 1:
                    src_a = acc.at[rows_a(send_a_c, k), :]
                    src_b = acc.at[rows_b(send_b_c, k), :]
                else:
                    src_a = out_ref.at[rows_a(send_a_c, k), :]
                    src_b = out_ref.at[rows_b(send_b_c, k), :]
                dst_a = out_ref.at[rows_a(send_a_c, k), :]
                dst_b = out_ref.at[rows_b(send_b_c, k), :]
            rdma_a = pltpu.make_async_remote_copy(
                src_ref=src_a,
                dst_ref=dst_a,
                send_sem=send_sem_a.at[slot, k],
                recv_sem=recv_sem_a.at[slot, k],
                device_id=(right,),
                device_id_type=pl.DeviceIdType.MESH,
            )
            rdma_b = pltpu.make_async_remote_copy(
                src_ref=src_b,
                dst_ref=dst_b,
                send_sem=send_sem_b.at[slot, k],
                recv_sem=recv_sem_b.at[slot, k],
                device_id=(left,),
                device_id_type=pl.DeviceIdType.MESH,
            )
            rdma_a.start()
            rdma_b.start()
            return rdma_a, rdma_b

        def grant(inc=1):
            pl.semaphore_signal(
                ready_a, inc=inc,
                device_id=(left,), device_id_type=pl.DeviceIdType.MESH,
            )
            pl.semaphore_signal(
                ready_b, inc=inc,
                device_id=(right,), device_id_type=pl.DeviceIdType.MESH,
            )

        grant(inc=2)

        gemm_half_a(my)
        gemm_half_b(my)
        pl.semaphore_wait(ready_a, 1)
        pl.semaphore_wait(ready_b, 1)
        inflight = {(0, k): issue(0, k) for k in range(K_SUB)}
        own_flush = []

        for s in range(N_STEP):
            ag = s >= N_DEV - 1
            _, recv_a_c, _, recv_b_c = chunk_ids(s)
            if not ag:
                gemm_half_a(recv_a_c)
                gemm_half_b(recv_b_c)
            if s + 1 < N_STEP:
                pl.semaphore_wait(ready_a, 1)
                pl.semaphore_wait(ready_b, 1)
            for k in range(K_SUB):
                rdma_a, rdma_b = inflight[(s, k)]
                rdma_a.wait_recv()
                if not ag:
                    ia = rows_a(recv_a_c, k)
                    acc[ia, :] = acc[ia, :] + recv_a[s % 2, k, :, :]
                rdma_b.wait_recv()
                if not ag:
                    ib = rows_b(recv_b_c, k)
                    acc[ib, :] = acc[ib, :] + recv_b[s % 2, k, :, :]
                if s == N_DEV - 2 and k == K_SUB - 1:
                    ca = pl.ds(lax.rem(my + 1, N_DEV) * hchunk, hchunk)
                    cpa = pltpu.make_async_copy(
                        acc.at[ca, :], out_ref.at[ca, :], own_sem.at[0]
                    )
                    cpa.start()
                    cb = pl.ds(
                        halfm + lax.rem(my - 1 + N_DEV, N_DEV) * hchunk, hchunk
                    )
                    cpb = pltpu.make_async_copy(
                        acc.at[cb, :], out_ref.at[cb, :], own_sem.at[1]
                    )
                    cpb.start()
                    own_flush = [cpa, cpb]
                if s + 1 < N_STEP:
                    if s >= 1:
                        pa, pb = inflight.pop((s - 1, k))
                        pa.wait_send()
                        pb.wait_send()
                    inflight[(s + 1, k)] = issue(s + 1, k)
            if s < N_STEP - 2:
                grant()

        for key in sorted(inflight):
            pa, pb = inflight.pop(key)
            pa.wait_send()
            pb.wait_send()
        for cp in own_flush:
            cp.wait()

    return pl.pallas_call(
        body,
        out_shape=jax.ShapeDtypeStruct((m, n), jnp.float32),
        in_specs=[
            pl.BlockSpec(memory_space=pltpu.VMEM),
            pl.BlockSpec(memory_space=pltpu.VMEM),
        ],
        out_specs=pl.BlockSpec(memory_space=pltpu.HBM),
        scratch_shapes=[
            pltpu.VMEM((m, n), jnp.float32),
            pltpu.VMEM((2, K_SUB, sub, n), jnp.float32),
            pltpu.VMEM((2, K_SUB, sub, n), jnp.float32),
            pltpu.SemaphoreType.DMA((2, K_SUB)),
            pltpu.SemaphoreType.DMA((2, K_SUB)),
            pltpu.SemaphoreType.DMA((2, K_SUB)),
            pltpu.SemaphoreType.DMA((2, K_SUB)),
            pltpu.SemaphoreType.REGULAR,
            pltpu.SemaphoreType.REGULAR,
            pltpu.SemaphoreType.DMA((2,)),
        ],
        compiler_params=pltpu.CompilerParams(
            vmem_limit_bytes=100 * 1024 * 1024,
        ),
    )(x, w_mat)


# device time: 351108 ns/iter; 1.0385x vs baseline; 1.0073x over previous
import jax
import jax.numpy as jnp
from jax import lax
from jax.experimental import pallas as pl
from jax.experimental.pallas import tpu as pltpu

N_DEV = 8
N_STEP = 2 * (N_DEV - 1)
K_SUB = 2


def kernel(x, w_mat):
    m, k_per = x.shape
    _, n = w_mat.shape
    halfm = m // 2
    assert halfm % N_DEV == 0
    hchunk = halfm // N_DEV
    assert hchunk % K_SUB == 0
    sub = hchunk // K_SUB

    def body(
        x_ref, w_ref, out_ref,
        acc, recv_a, recv_b,
        send_sem_a, recv_sem_a, send_sem_b, recv_sem_b,
        ready_a, ready_b, own_sem,
    ):
        my = lax.axis_index("i")
        left = lax.rem(my - 1 + N_DEV, N_DEV)
        right = lax.rem(my + 1, N_DEV)

        def rows_a(c, k):
            return pl.ds(c * hchunk + k * sub, sub)

        def rows_b(c, k):
            return pl.ds(halfm + c * hchunk + k * sub, sub)

        def gemm_rows(i):
            acc[i, :] = jnp.dot(
                x_ref[i, :], w_ref[:, :], preferred_element_type=jnp.float32
            )

        def gemm_half_a(c):
            gemm_rows(pl.ds(c * hchunk, hchunk))

        def gemm_half_b(c):
            gemm_rows(pl.ds(halfm + c * hchunk, hchunk))

        def chunk_ids(s):
            if s < N_DEV - 1:
                return (
                    lax.rem(my - s + N_DEV, N_DEV),
                    lax.rem(my - s - 1 + 2 * N_DEV, N_DEV),
                    lax.rem(my + s, N_DEV),
                    lax.rem(my + s + 1, N_DEV),
                )
            t = s - (N_DEV - 1)
            return (
                lax.rem(my + 1 - t + 2 * N_DEV, N_DEV),
                lax.rem(my - t + 2 * N_DEV, N_DEV),
                lax.rem(my - 1 + t + N_DEV, N_DEV),
                lax.rem(my + t, N_DEV),
            )

        def issue(s, k):
            slot = s % 2
            send_a_c, _, send_b_c, _ = chunk_ids(s)
            if s < N_DEV - 1:
                src_a = acc.at[rows_a(send_a_c, k), :]
                src_b = acc.at[rows_b(send_b_c, k), :]
                dst_a = recv_a.at[slot, k]
                dst_b = recv_b.at[slot, k]
            else:
                if s == N_DEV - 1:
                    src_a = acc.at[rows_a(send_a_c, k), :]
                    src_b = acc.at[rows_b(send_b_c, k), :]
                else:
                    src_a = out_ref.at[rows_a(send_a_c, k), :]
                    src_b = out_ref.at[rows_b(send_b_c, k), :]
                dst_a = out_ref.at[rows_a(send_a_c, k), :]
                dst_b = out_ref.at[rows_b(send_b_c, k), :]
            rdma_a = pltpu.make_async_remote_copy(
                src_ref=src_a,
                dst_ref=dst_a,
                send_sem=send_sem_a.at[slot, k],
                recv_sem=recv_sem_a.at[slot, k],
                device_id=(right,),
                device_id_type=pl.DeviceIdType.MESH,
            )
            rdma_b = pltpu.make_async_remote_copy(
                src_ref=src_b,
                dst_ref=dst_b,
                send_sem=send_sem_b.at[slot, k],
                recv_sem=recv_sem_b.at[slot, k],
                device_id=(left,),
                device_id_type=pl.DeviceIdType.MESH,
            )
            rdma_a.start()
            rdma_b.start()
            return rdma_a, rdma_b

        def grant(inc=1):
            pl.semaphore_signal(
                ready_a, inc=inc,
                device_id=(left,), device_id_type=pl.DeviceIdType.MESH,
            )
            pl.semaphore_signal(
                ready_b, inc=inc,
                device_id=(right,), device_id_type=pl.DeviceIdType.MESH,
            )

        barrier_sem = pltpu.get_barrier_semaphore()
        for nbr in (left, right):
            pl.semaphore_signal(
                barrier_sem, inc=1,
                device_id=(nbr,), device_id_type=pl.DeviceIdType.MESH,
            )
        pl.semaphore_wait(barrier_sem, 2)

        grant(inc=2)

        inflight = {}
        own_flush = []
        pl.semaphore_wait(ready_a, 1)
        pl.semaphore_wait(ready_b, 1)
        for k in range(K_SUB):
            gemm_rows(rows_a(my, k))
            gemm_rows(rows_b(my, k))
            inflight[(0, k)] = issue(0, k)

        for s in range(N_STEP):
            ag = s >= N_DEV - 1
            _, recv_a_c, _, recv_b_c = chunk_ids(s)
            if not ag:
                gemm_half_a(recv_a_c)
                gemm_half_b(recv_b_c)
            if s + 1 < N_STEP:
                pl.semaphore_wait(ready_a, 1)
                pl.semaphore_wait(ready_b, 1)
            for k in range(K_SUB):
                rdma_a, rdma_b = inflight[(s, k)]
                rdma_a.wait_recv()
                if not ag:
                    ia = rows_a(recv_a_c, k)
                    acc[ia, :] = acc[ia, :] + recv_a[s % 2, k, :, :]
                rdma_b.wait_recv()
                if not ag:
                    ib = rows_b(recv_b_c, k)
                    acc[ib, :] = acc[ib, :] + recv_b[s % 2, k, :, :]
                if s == N_DEV - 2 and k == K_SUB - 1:
                    ca = pl.ds(lax.rem(my + 1, N_DEV) * hchunk, hchunk)
                    cpa = pltpu.make_async_copy(
                        acc.at[ca, :], out_ref.at[ca, :], own_sem.at[0]
                    )
                    cpa.start()
                    cb = pl.ds(
                        halfm + lax.rem(my - 1 + N_DEV, N_DEV) * hchunk, hchunk
                    )
                    cpb = pltpu.make_async_copy(
                        acc.at[cb, :], out_ref.at[cb, :], own_sem.at[1]
                    )
                    cpb.start()
                    own_flush = [cpa, cpb]
                if s + 1 < N_STEP:
                    if s >= 1:
                        pa, pb = inflight.pop((s - 1, k))
                        pa.wait_send()
                        pb.wait_send()
                    inflight[(s + 1, k)] = issue(s + 1, k)
            if s < N_STEP - 2:
                grant()

        for key in sorted(inflight):
            pa, pb = inflight.pop(key)
            pa.wait_send()
            pb.wait_send()
        for cp in own_flush:
            cp.wait()

    return pl.pallas_call(
        body,
        out_shape=jax.ShapeDtypeStruct((m, n), jnp.float32),
        in_specs=[
            pl.BlockSpec(memory_space=pltpu.VMEM),
            pl.BlockSpec(memory_space=pltpu.VMEM),
        ],
        out_specs=pl.BlockSpec(memory_space=pltpu.HBM),
        scratch_shapes=[
            pltpu.VMEM((m, n), jnp.float32),
            pltpu.VMEM((2, K_SUB, sub, n), jnp.float32),
            pltpu.VMEM((2, K_SUB, sub, n), jnp.float32),
            pltpu.SemaphoreType.DMA((2, K_SUB)),
            pltpu.SemaphoreType.DMA((2, K_SUB)),
            pltpu.SemaphoreType.DMA((2, K_SUB)),
            pltpu.SemaphoreType.DMA((2, K_SUB)),
            pltpu.SemaphoreType.REGULAR,
            pltpu.SemaphoreType.REGULAR,
            pltpu.SemaphoreType.DMA((2,)),
        ],
        compiler_params=pltpu.CompilerParams(
            vmem_limit_bytes=100 * 1024 * 1024,
            collective_id=0,
        ),
    )(x, w_mat)


# device time: 260740 ns/iter; 1.3985x vs baseline; 1.3466x over previous
import jax
import jax.numpy as jnp
from jax import lax
from jax.experimental import pallas as pl
from jax.experimental.pallas import tpu as pltpu

N_DEV = 8

PART_ROWS = (1408, 1344, 1344)
PART_DIMS = ((0, 1, 2), (1, 2, 0), (2, 0, 1))
PHASE_SUBS = (4, 2, 1, 1, 2, 4)
SEM_OFF = (0, 4, 6, 7, 8, 10)
N_SEM = 14


def kernel(x, w_mat):
    m, k_per = x.shape
    _, n = w_mat.shape
    assert m == sum(PART_ROWS)
    base = (0, PART_ROWS[0], PART_ROWS[0] + PART_ROWS[1])
    scr_rows = tuple(P // 2 for P in PART_ROWS)
    scr_base = (0, scr_rows[0], scr_rows[0] + scr_rows[1])
    tot_scr = sum(scr_rows)

    def body(
        x_ref, w_ref, out_ref,
        acc, scr,
        sems_s0, sems_r0, sems_s1, sems_r1, sems_s2, sems_r2,
        flush_sem,
    ):
        my = lax.axis_index("i")

        def coords_of(i):
            z = i // 4
            r = i - 4 * z
            y = r // 2
            xx = jnp.where((r == 1) | (r == 2), 1, 0)
            return (xx, y, z)

        def id_of(c):
            xx, y, z = c
            return xx + y * (3 - 2 * xx) + 4 * z

        my_c = coords_of(my)

        def partner_id(d):
            c = list(my_c)
            c[d] = 1 - c[d]
            return id_of(tuple(c))

        partners = [partner_id(d) for d in range(3)]
        send_sems = (sems_s0, sems_s1, sems_s2)
        recv_sems = (sems_r0, sems_r1, sems_r2)

        parts = []
        for p in range(3):
            P = PART_ROWS[p]
            d0, d1, d2 = PART_DIMS[p]
            bits = (my_c[d0], my_c[d1], my_c[d2])
            W = [jnp.int32(base[p])]
            keep = []
            send = []
            for j in range(3):
                half = P >> (j + 1)
                keep_off = W[j] + bits[j] * half
                send_off = W[j] + (1 - bits[j]) * half
                keep.append(keep_off)
                send.append(send_off)
                W.append(keep_off)
            parts.append({
                "P": P,
                "dims": (d0, d1, d2),
                "W": W,
                "keep": keep,
                "send": send,
                "rz": (scr_base[p], send[0], send[0] + P // 4),
            })

        def gemm_rows(off, nrows):
            i = pl.ds(off, nrows)
            acc[i, :] = jnp.dot(
                x_ref[i, :], w_ref[:, :], preferred_element_type=jnp.float32
            )

        barrier_sem = pltpu.get_barrier_semaphore()
        for d in range(3):
            pl.semaphore_signal(
                barrier_sem, inc=1,
                device_id=(partners[d],), device_id_type=pl.DeviceIdType.MESH,
            )
        pl.semaphore_wait(barrier_sem, 3)

        all_rdmas = []

        def exchange(p, phase, k, src_ref, src_off, dst_ref, dst_off, nrows):
            d = parts[p]["dims"][phase if phase < 3 else 5 - phase]
            si = SEM_OFF[phase] + k
            rdma = pltpu.make_async_remote_copy(
                src_ref=src_ref.at[pl.ds(src_off, nrows), :],
                dst_ref=dst_ref.at[pl.ds(dst_off, nrows), :],
                send_sem=send_sems[p].at[si],
                recv_sem=recv_sems[p].at[si],
                device_id=(partners[d],),
                device_id_type=pl.DeviceIdType.MESH,
            )
            rdma.start()
            all_rdmas.append(rdma)
            return rdma

        rs = {}
        for k in range(PHASE_SUBS[0]):
            for p in range(3):
                sub = (parts[p]["P"] // 2) // PHASE_SUBS[0]
                off = parts[p]["send"][0] + k * sub
                gemm_rows(off, sub)
                rs[(p, 0, k)] = exchange(
                    p, 0, k, acc, off, scr, parts[p]["rz"][0] + k * sub, sub
                )
        for k in range(PHASE_SUBS[0]):
            for p in range(3):
                sub = (parts[p]["P"] // 2) // PHASE_SUBS[0]
                gemm_rows(parts[p]["keep"][0] + k * sub, sub)

        for j in range(3):
            for k in range(PHASE_SUBS[j]):
                for p in range(3):
                    half = parts[p]["P"] >> (j + 1)
                    sub = half // PHASE_SUBS[j]
                    rs[(p, j, k)].wait_recv()
                    ia = pl.ds(parts[p]["keep"][j] + k * sub, sub)
                    isc = pl.ds(parts[p]["rz"][j] + k * sub, sub)
                    rz_ref = scr if j == 0 else acc
                    acc[ia, :] = acc[ia, :] + rz_ref[isc, :]
                    if k == PHASE_SUBS[j] - 1 and j < 2:
                        nhalf = parts[p]["P"] >> (j + 2)
                        nsub = nhalf // PHASE_SUBS[j + 1]
                        for k2 in range(PHASE_SUBS[j + 1]):
                            rs[(p, j + 1, k2)] = exchange(
                                p, j + 1, k2,
                                acc, parts[p]["send"][j + 1] + k2 * nsub,
                                acc, parts[p]["rz"][j + 1] + k2 * nsub,
                                nsub,
                            )

        flushes = []
        ag = {}
        for p in range(3):
            blk = parts[p]["P"] // 8
            w3 = parts[p]["W"][3]
            cp = pltpu.make_async_copy(
                acc.at[pl.ds(w3, blk), :],
                out_ref.at[pl.ds(w3, blk), :],
                flush_sem.at[p],
            )
            cp.start()
            flushes.append(cp)
            ag[(p, 3, 0)] = exchange(p, 3, 0, acc, w3, out_ref, w3, blk)
        for p in range(3):
            ag[(p, 3, 0)].wait_recv()
            flushes[p].wait()
            quart = parts[p]["P"] // 4
            sub = quart // PHASE_SUBS[4]
            for k in range(PHASE_SUBS[4]):
                off = parts[p]["W"][2] + k * sub
                ag[(p, 4, k)] = exchange(p, 4, k, out_ref, off, out_ref, off, sub)
        for k in range(PHASE_SUBS[4]):
            for p in range(3):
                ag[(p, 4, k)].wait_recv()
        for p in range(3):
            half = parts[p]["P"] // 2
            sub = half // PHASE_SUBS[5]
            for k in range(PHASE_SUBS[5]):
                off = parts[p]["W"][1] + k * sub
                ag[(p, 5, k)] = exchange(p, 5, k, out_ref, off, out_ref, off, sub)
        for k in range(PHASE_SUBS[5]):
            for p in range(3):
                ag[(p, 5, k)].wait_recv()

        for rdma in all_rdmas:
            rdma.wait_send()

    return pl.pallas_call(
        body,
        out_shape=jax.ShapeDtypeStruct((m, n), jnp.float32),
        in_specs=[
            pl.BlockSpec(memory_space=pltpu.VMEM),
            pl.BlockSpec(memory_space=pltpu.VMEM),
        ],
        out_specs=pl.BlockSpec(memory_space=pltpu.HBM),
        scratch_shapes=[
            pltpu.VMEM((m, n), jnp.float32),
            pltpu.VMEM((tot_scr, n), jnp.float32),
            pltpu.SemaphoreType.DMA((N_SEM,)),
            pltpu.SemaphoreType.DMA((N_SEM,)),
            pltpu.SemaphoreType.DMA((N_SEM,)),
            pltpu.SemaphoreType.DMA((N_SEM,)),
            pltpu.SemaphoreType.DMA((N_SEM,)),
            pltpu.SemaphoreType.DMA((N_SEM,)),
            pltpu.SemaphoreType.DMA((3,)),
        ],
        compiler_params=pltpu.CompilerParams(
            vmem_limit_bytes=100 * 1024 * 1024,
            collective_id=0,
        ),
    )(x, w_mat)
